# Initial kernel scaffold; baseline (speedup 1.0000x reference)
import jax
import jax.numpy as jnp
from jax.experimental import pallas as pl


def kernel(x, w, mu, sigma):
    raise NotImplementedError("write your pallas kernel here")


# speedup vs baseline: 2.2870x; 2.2870x over previous
import functools
import math

import jax
import jax.numpy as jnp
import numpy as np
from jax.experimental import pallas as pl
from jax.experimental.pallas import tpu as pltpu

_KPAD = 128
_NEG = -1e30


def _tc_body(x_ref, a_ref, b_ref, c_ref, out_ref):
    x = x_ref[...]
    comps = (
        jnp.dot(x * x, a_ref[...], preferred_element_type=jnp.float32)
        + jnp.dot(x, b_ref[...], preferred_element_type=jnp.float32)
        + c_ref[...]
    )
    m = jnp.max(comps, axis=1, keepdims=True)
    s = jnp.sum(jnp.exp(comps - m), axis=1, keepdims=True)
    out_ref[...] = (m + jnp.log(s))[:, 0]


def kernel(x, w, mu, sigma):
    N, D = x.shape
    K = w.shape[0]
    inv2 = 1.0 / (sigma * sigma)
    A = jnp.zeros((D, _KPAD), jnp.float32).at[:, :K].set((-0.5 * inv2).T)
    B = jnp.zeros((D, _KPAD), jnp.float32).at[:, :K].set((mu * inv2).T)
    c_k = (
        jnp.log(w[:, 0])
        - 0.5 * (D * math.log(2.0 * math.pi)
                 + jnp.sum(jnp.log(sigma * sigma), axis=1)
                 + jnp.sum(mu * mu * inv2, axis=1))
    )
    c = jnp.full((1, _KPAD), _NEG, jnp.float32).at[0, :K].set(c_k)

    BN = 2048
    grid = (N // BN,)
    return pl.pallas_call(
        _tc_body,
        grid=grid,
        in_specs=[
            pl.BlockSpec((BN, D), lambda i: (i, 0)),
            pl.BlockSpec((D, _KPAD), lambda i: (0, 0)),
            pl.BlockSpec((D, _KPAD), lambda i: (0, 0)),
            pl.BlockSpec((1, _KPAD), lambda i: (0, 0)),
        ],
        out_specs=pl.BlockSpec((BN,), lambda i: (i,)),
        out_shape=jax.ShapeDtypeStruct((N,), jnp.float32),
        compiler_params=pltpu.CompilerParams(
            dimension_semantics=("arbitrary",),
        ),
    )(x, A, B, c)
